# baseline (speedup 1.0000x reference)
import functools

import jax
import jax.numpy as jnp
from jax import lax
from jax.experimental import pallas as pl
from jax.experimental.pallas import tpu as pltpu
from jax.experimental.pallas import tpu_sc as plsc

B, N, P, D = 16, 512, 128, 768
DEPTH, H = 2, 12
HD = D // H
NUM_MASKED = N // 2
L = 16
NCHUNK = N // L


def _sc_mask_seg(rand_scores, valid_length):
    mesh = plsc.VectorSubcoreMesh(core_axis_name="c", subcore_axis_name="s")

    @functools.partial(
        pl.kernel,
        out_type=(
            jax.ShapeDtypeStruct((B, N), jnp.float32),
            jax.ShapeDtypeStruct((B, N), jnp.float32),
        ),
        mesh=mesh,
        compiler_params=pltpu.CompilerParams(needs_layout_passes=False),
        scratch_types=[
            pltpu.VMEM((N,), jnp.float32),
            pltpu.VMEM((N,), jnp.float32),
            pltpu.VMEM((N,), jnp.float32),
            pltpu.VMEM((B,), jnp.int32),
        ],
    )
    def sc_kernel(scores_hbm, vl_hbm, mask_hbm, seg_hbm, row_v, mask_v, seg_v, vl_v):
        cid = lax.axis_index("c")
        sid = lax.axis_index("s")
        wid = sid * 2 + cid

        @pl.when(wid < B)
        def _():
            pltpu.sync_copy(scores_hbm.at[wid], row_v)
            pltpu.sync_copy(vl_hbm, vl_v)

            lane = lax.iota(jnp.int32, L)
            popc = plsc.all_reduce_population_count

            vlv = vl_v[...]
            is_me = lane == wid
            vl_splat = jnp.zeros((L,), jnp.int32)
            for kbit in range(12):
                hit = is_me & (((vlv >> kbit) & 1) == 1)
                vl_splat = vl_splat + (popc(hit) << kbit)

            def count_ge(t_splat):
                def body(ch, acc):
                    v = row_v[pl.ds(ch * L, L)]
                    vi = plsc.bitcast(v, jnp.int32)
                    return acc + popc(vi >= t_splat)

                return lax.fori_loop(0, NCHUNK, body, jnp.zeros((L,), jnp.int32))

            def bis(_, carry):
                lo, hi = carry
                mid = (lo + hi) >> 1
                big = count_ge(mid) >= NUM_MASKED
                return (jnp.where(big, mid, lo), jnp.where(big, hi, mid))

            v_bits, _ = lax.fori_loop(
                0,
                31,
                bis,
                (jnp.zeros((L,), jnp.int32), jnp.full((L,), 0x3F800000, jnp.int32)),
            )
            need_eq = NUM_MASKED - count_ge(v_bits + 1)

            def cnt_le(p_splat):
                def body(ch, acc):
                    v = row_v[pl.ds(ch * L, L)]
                    vi = plsc.bitcast(v, jnp.int32)
                    idx0 = lane + ch * L
                    return acc + popc((vi == v_bits) & (idx0 <= p_splat))

                return lax.fori_loop(0, NCHUNK, body, jnp.zeros((L,), jnp.int32))

            def bis2(_, carry):
                lo, hi = carry
                mid = (lo + hi) >> 1
                enough = cnt_le(mid) >= need_eq
                return (jnp.where(enough, lo, mid), jnp.where(enough, mid, hi))

            _, p_star = lax.fori_loop(
                0,
                10,
                bis2,
                (jnp.full((L,), -1, jnp.int32), jnp.full((L,), N - 1, jnp.int32)),
            )

            def fill(ch, carry):
                v = row_v[pl.ds(ch * L, L)]
                vi = plsc.bitcast(v, jnp.int32)
                idx0 = lane + ch * L
                gt = vi > v_bits
                take = (vi == v_bits) & (idx0 <= p_star)
                mask_v[pl.ds(ch * L, L)] = jnp.where(gt | take, 1.0, 0.0)
                seg_v[pl.ds(ch * L, L)] = jnp.where(
                    5 * (idx0 + 1) <= vl_splat, 1.0, 0.0
                )
                return carry

            lax.fori_loop(0, NCHUNK, fill, jnp.int32(0))
            pltpu.sync_copy(mask_v, mask_hbm.at[wid])
            pltpu.sync_copy(seg_v, seg_hbm.at[wid])

    return sc_kernel(rand_scores, valid_length)


def _ln(x, g, b):
    m = jnp.mean(x, axis=-1, keepdims=True)
    v = jnp.mean(x * x, axis=-1, keepdims=True) - m * m
    return (x - m) * lax.rsqrt(v + 1e-5) * g + b


def _dotT(a, wt):
    return lax.dot_general(
        a, wt, (((1,), (0,)), ((), ())), preferred_element_type=jnp.float32
    )


def _softmax_unnorm(s):
    e = jnp.exp(s)
    return e, 1.0 / jnp.sum(e, axis=-1, keepdims=True)


def _layer(x, Wqkv_ref, bqkv_ref, Wo_ref, bo_ref, g1_ref, b1_ref, g2_ref,
           b2_ref, W1_ref, b1m_ref, W2_ref, b2m_ref):
    h = _ln(x, g1_ref[...], b1_ref[...])
    qkv = _dotT(h, Wqkv_ref[...]) + bqkv_ref[...]
    heads = []
    for hh in range(H):
        q = qkv[:, hh * HD : (hh + 1) * HD]
        k = qkv[:, D + hh * HD : D + (hh + 1) * HD]
        v = qkv[:, 2 * D + hh * HD : 2 * D + (hh + 1) * HD]
        s = lax.dot_general(
            q, k, (((1,), (1,)), ((), ())),
            preferred_element_type=jnp.float32,
        ) * (1.0 / 8.0)
        e, r = _softmax_unnorm(s)
        ev = lax.dot_general(
            e, v, (((1,), (0,)), ((), ())),
            preferred_element_type=jnp.float32,
        )
        heads.append(ev * r)
    o = jnp.concatenate(heads, axis=1)
    x = x + _dotT(o, Wo_ref[...]) + bo_ref[...]
    h2 = _ln(x, g2_ref[...], b2_ref[...])
    m1 = jax.nn.gelu(_dotT(h2, W1_ref[...]) + b1m_ref[...])
    return x + _dotT(m1, W2_ref[...]) + b2m_ref[...]


def _tc_body1(
    img_ref, mask_ref, seg_ref, Wp_ref, bp_ref, pos_ref, val_ref, mtok_ref,
    Wqkv_ref, bqkv_ref, Wo_ref, bo_ref, g1_ref, b1_ref, g2_ref, b2_ref,
    W1_ref, b1m_ref, W2_ref, b2m_ref, x_out_ref,
):
    img = img_ref[0]
    msk = mask_ref[0]
    seg = seg_ref[0]

    val0 = val_ref[0:1, :]
    val1 = val_ref[1:2, :]
    posb = pos_ref[...] + val0 + seg * (val1 - val0)
    tokens = _dotT(img, Wp_ref[...]) + bp_ref[...] + posb
    mtok = mtok_ref[...] + posb
    x = jnp.where(msk != 0.0, mtok, tokens)

    x_out_ref[0] = _layer(
        x, Wqkv_ref, bqkv_ref, Wo_ref, bo_ref, g1_ref, b1_ref, g2_ref,
        b2_ref, W1_ref, b1m_ref, W2_ref, b2m_ref,
    )


def _tc_body2(
    x_ref, img_ref, mask_ref, Wt_ref, bt_ref,
    Wqkv_ref, bqkv_ref, Wo_ref, bo_ref, g1_ref, b1_ref, g2_ref, b2_ref,
    W1_ref, b1m_ref, W2_ref, b2m_ref, out_ref,
):
    x = _layer(
        x_ref[0], Wqkv_ref, bqkv_ref, Wo_ref, bo_ref, g1_ref, b1_ref,
        g2_ref, b2_ref, W1_ref, b1m_ref, W2_ref, b2m_ref,
    )
    pred = _dotT(x, Wt_ref[...]) + bt_ref[...]
    contrib = jnp.sum(jnp.abs(pred - img_ref[0]) * mask_ref[0])
    out_ref[...] = jnp.reshape(contrib, (1, 1, 1))


def _full(arr):
    return pl.BlockSpec(arr.shape, lambda b: (0,) * arr.ndim)


def _batched(arr):
    return pl.BlockSpec(
        (1,) + arr.shape[1:], lambda b: (b,) + (0,) * (arr.ndim - 1)
    )


def _layer_ops(lyr):
    return (
        lyr["Wqkv"].T, lyr["bqkv"].reshape(1, 3 * D), lyr["Wo"].T,
        lyr["bo"].reshape(1, D), lyr["g1"].reshape(1, D),
        lyr["b1"].reshape(1, D), lyr["g2"].reshape(1, D),
        lyr["b2"].reshape(1, D), lyr["W1"].T, lyr["b1m"].reshape(1, 4 * D),
        lyr["W2"].T, lyr["b2m"].reshape(1, D),
    )


def kernel(img, valid_length, rand_scores, Wp, bp, pos_table, val_table,
           mask_token, Wt, bt, layers):
    valid_length = valid_length.astype(jnp.int32)
    mask, seg = _sc_mask_seg(rand_scores, valid_length)

    pos = pos_table[1 : N + 1]
    mask3 = mask[..., None]
    seg3 = seg[..., None]

    ops1 = (
        img, mask3, seg3, Wp.T, bp.reshape(1, D), pos, val_table,
        mask_token.reshape(1, D),
    ) + _layer_ops(layers[0])
    in_specs1 = [_batched(img), _batched(mask3), _batched(seg3)] + [
        _full(a) for a in ops1[3:]
    ]
    x1 = pl.pallas_call(
        _tc_body1,
        grid=(B,),
        in_specs=in_specs1,
        out_specs=pl.BlockSpec((1, N, D), lambda b: (b, 0, 0)),
        out_shape=jax.ShapeDtypeStruct((B, N, D), jnp.float32),
        compiler_params=pltpu.CompilerParams(
            dimension_semantics=("arbitrary",),
        ),
    )(*ops1)

    ops2 = (x1, img, mask3, Wt.T, bt.reshape(1, P)) + _layer_ops(layers[1])
    in_specs2 = [_batched(x1), _batched(img), _batched(mask3)] + [
        _full(a) for a in ops2[3:]
    ]
    partials = pl.pallas_call(
        _tc_body2,
        grid=(B,),
        in_specs=in_specs2,
        out_specs=pl.BlockSpec((1, 1, 1), lambda b: (b, 0, 0)),
        out_shape=jax.ShapeDtypeStruct((B, 1, 1), jnp.float32),
        compiler_params=pltpu.CompilerParams(
            dimension_semantics=("arbitrary",),
        ),
    )(*ops2)

    return jnp.sum(partials) / (B * NUM_MASKED * P) / NUM_MASKED


# speedup vs baseline: 1.0495x; 1.0495x over previous
import functools

import jax
import jax.numpy as jnp
from jax import lax
from jax.experimental import pallas as pl
from jax.experimental.pallas import tpu as pltpu
from jax.experimental.pallas import tpu_sc as plsc

B, N, P, D = 16, 512, 128, 768
DEPTH, H = 2, 12
HD = D // H
NUM_MASKED = N // 2
L = 16
NCHUNK = N // L


def _sc_mask_seg(rand_scores, valid_length):
    mesh = plsc.VectorSubcoreMesh(core_axis_name="c", subcore_axis_name="s")

    @functools.partial(
        pl.kernel,
        out_type=(
            jax.ShapeDtypeStruct((B, N), jnp.float32),
            jax.ShapeDtypeStruct((B, N), jnp.float32),
        ),
        mesh=mesh,
        compiler_params=pltpu.CompilerParams(needs_layout_passes=False),
        scratch_types=[
            pltpu.VMEM((N,), jnp.float32),
            pltpu.VMEM((N,), jnp.float32),
            pltpu.VMEM((N,), jnp.float32),
            pltpu.VMEM((B,), jnp.int32),
        ],
    )
    def sc_kernel(scores_hbm, vl_hbm, mask_hbm, seg_hbm, row_v, mask_v, seg_v, vl_v):
        cid = lax.axis_index("c")
        sid = lax.axis_index("s")
        wid = sid * 2 + cid

        @pl.when(wid < B)
        def _():
            pltpu.sync_copy(scores_hbm.at[wid], row_v)
            pltpu.sync_copy(vl_hbm, vl_v)

            lane = lax.iota(jnp.int32, L)
            popc = plsc.all_reduce_population_count

            vlv = vl_v[...]
            is_me = lane == wid
            vl_splat = jnp.zeros((L,), jnp.int32)
            for kbit in range(12):
                hit = is_me & (((vlv >> kbit) & 1) == 1)
                vl_splat = vl_splat + (popc(hit) << kbit)

            def count_ge(t_splat):
                def body(ch, acc):
                    v = row_v[pl.ds(ch * L, L)]
                    vi = plsc.bitcast(v, jnp.int32)
                    return acc + popc(vi >= t_splat)

                return lax.fori_loop(0, NCHUNK, body, jnp.zeros((L,), jnp.int32))

            def bis(_, carry):
                lo, hi = carry
                mid = (lo + hi) >> 1
                big = count_ge(mid) >= NUM_MASKED
                return (jnp.where(big, mid, lo), jnp.where(big, hi, mid))

            v_bits, _ = lax.fori_loop(
                0,
                31,
                bis,
                (jnp.zeros((L,), jnp.int32), jnp.full((L,), 0x3F800000, jnp.int32)),
            )
            need_eq = NUM_MASKED - count_ge(v_bits + 1)

            def cnt_le(p_splat):
                def body(ch, acc):
                    v = row_v[pl.ds(ch * L, L)]
                    vi = plsc.bitcast(v, jnp.int32)
                    idx0 = lane + ch * L
                    return acc + popc((vi == v_bits) & (idx0 <= p_splat))

                return lax.fori_loop(0, NCHUNK, body, jnp.zeros((L,), jnp.int32))

            def bis2(_, carry):
                lo, hi = carry
                mid = (lo + hi) >> 1
                enough = cnt_le(mid) >= need_eq
                return (jnp.where(enough, lo, mid), jnp.where(enough, mid, hi))

            _, p_star = lax.fori_loop(
                0,
                10,
                bis2,
                (jnp.full((L,), -1, jnp.int32), jnp.full((L,), N - 1, jnp.int32)),
            )

            def fill(ch, carry):
                v = row_v[pl.ds(ch * L, L)]
                vi = plsc.bitcast(v, jnp.int32)
                idx0 = lane + ch * L
                gt = vi > v_bits
                take = (vi == v_bits) & (idx0 <= p_star)
                mask_v[pl.ds(ch * L, L)] = jnp.where(gt | take, 1.0, 0.0)
                seg_v[pl.ds(ch * L, L)] = jnp.where(
                    5 * (idx0 + 1) <= vl_splat, 1.0, 0.0
                )
                return carry

            lax.fori_loop(0, NCHUNK, fill, jnp.int32(0))
            pltpu.sync_copy(mask_v, mask_hbm.at[wid])
            pltpu.sync_copy(seg_v, seg_hbm.at[wid])

    return sc_kernel(rand_scores, valid_length)


def _ln(x, g, b):
    m = jnp.mean(x, axis=-1, keepdims=True)
    v = jnp.mean(x * x, axis=-1, keepdims=True) - m * m
    return (x - m) * lax.rsqrt(v + 1e-5) * g + b


def _dotT(a, w):
    return lax.dot_general(
        a, w, (((1,), (1,)), ((), ())), preferred_element_type=jnp.float32
    )


def _softmax_unnorm(s):
    e = jnp.exp(s)
    return e, 1.0 / jnp.sum(e, axis=-1, keepdims=True)


def _layer(x, Wqkv_ref, bqkv_ref, Wo_ref, bo_ref, g1_ref, b1_ref, g2_ref,
           b2_ref, W1_ref, b1m_ref, W2_ref, b2m_ref):
    h = _ln(x, g1_ref[...], b1_ref[...])
    qkv = _dotT(h, Wqkv_ref[...]) + bqkv_ref[...]
    heads = []
    for hh in range(H):
        q = qkv[:, hh * HD : (hh + 1) * HD]
        k = qkv[:, D + hh * HD : D + (hh + 1) * HD]
        v = qkv[:, 2 * D + hh * HD : 2 * D + (hh + 1) * HD]
        s = lax.dot_general(
            q, k, (((1,), (1,)), ((), ())),
            preferred_element_type=jnp.float32,
        ) * (1.0 / 8.0)
        e, r = _softmax_unnorm(s)
        ev = lax.dot_general(
            e, v, (((1,), (0,)), ((), ())),
            preferred_element_type=jnp.float32,
        )
        heads.append(ev * r)
    o = jnp.concatenate(heads, axis=1)
    x = x + _dotT(o, Wo_ref[...]) + bo_ref[...]
    h2 = _ln(x, g2_ref[...], b2_ref[...])
    m1 = jax.nn.gelu(_dotT(h2, W1_ref[...]) + b1m_ref[...])
    return x + _dotT(m1, W2_ref[...]) + b2m_ref[...]


def _tc_body1(
    img_ref, mask_ref, seg_ref, Wp_ref, bp_ref, pos_ref, val_ref, mtok_ref,
    Wqkv_ref, bqkv_ref, Wo_ref, bo_ref, g1_ref, b1_ref, g2_ref, b2_ref,
    W1_ref, b1m_ref, W2_ref, b2m_ref, x_out_ref,
):
    img = img_ref[0]
    msk = mask_ref[0]
    seg = seg_ref[0]

    val0 = val_ref[0:1, :]
    val1 = val_ref[1:2, :]
    posb = pos_ref[...] + val0 + seg * (val1 - val0)
    tokens = _dotT(img, Wp_ref[...]) + bp_ref[...] + posb
    mtok = mtok_ref[...] + posb
    x = jnp.where(msk != 0.0, mtok, tokens)

    x_out_ref[0] = _layer(
        x, Wqkv_ref, bqkv_ref, Wo_ref, bo_ref, g1_ref, b1_ref, g2_ref,
        b2_ref, W1_ref, b1m_ref, W2_ref, b2m_ref,
    )


def _tc_body2(
    x_ref, img_ref, mask_ref, Wt_ref, bt_ref,
    Wqkv_ref, bqkv_ref, Wo_ref, bo_ref, g1_ref, b1_ref, g2_ref, b2_ref,
    W1_ref, b1m_ref, W2_ref, b2m_ref, out_ref,
):
    x = _layer(
        x_ref[0], Wqkv_ref, bqkv_ref, Wo_ref, bo_ref, g1_ref, b1_ref,
        g2_ref, b2_ref, W1_ref, b1m_ref, W2_ref, b2m_ref,
    )
    pred = _dotT(x, Wt_ref[...]) + bt_ref[...]
    contrib = jnp.sum(jnp.abs(pred - img_ref[0]) * mask_ref[0])
    out_ref[...] = jnp.reshape(contrib, (1, 1, 1))


def _full(arr):
    return pl.BlockSpec(arr.shape, lambda b: (0,) * arr.ndim)


def _batched(arr):
    return pl.BlockSpec(
        (1,) + arr.shape[1:], lambda b: (b,) + (0,) * (arr.ndim - 1)
    )


def _layer_ops(lyr):
    return (
        lyr["Wqkv"], lyr["bqkv"].reshape(1, 3 * D), lyr["Wo"],
        lyr["bo"].reshape(1, D), lyr["g1"].reshape(1, D),
        lyr["b1"].reshape(1, D), lyr["g2"].reshape(1, D),
        lyr["b2"].reshape(1, D), lyr["W1"], lyr["b1m"].reshape(1, 4 * D),
        lyr["W2"], lyr["b2m"].reshape(1, D),
    )


def kernel(img, valid_length, rand_scores, Wp, bp, pos_table, val_table,
           mask_token, Wt, bt, layers):
    valid_length = valid_length.astype(jnp.int32)
    mask, seg = _sc_mask_seg(rand_scores, valid_length)

    pos = pos_table[1 : N + 1]
    mask3 = mask[..., None]
    seg3 = seg[..., None]

    ops1 = (
        img, mask3, seg3, Wp, bp.reshape(1, D), pos, val_table,
        mask_token.reshape(1, D),
    ) + _layer_ops(layers[0])
    in_specs1 = [_batched(img), _batched(mask3), _batched(seg3)] + [
        _full(a) for a in ops1[3:]
    ]
    x1 = pl.pallas_call(
        _tc_body1,
        grid=(B,),
        in_specs=in_specs1,
        out_specs=pl.BlockSpec((1, N, D), lambda b: (b, 0, 0)),
        out_shape=jax.ShapeDtypeStruct((B, N, D), jnp.float32),
        compiler_params=pltpu.CompilerParams(
            dimension_semantics=("arbitrary",),
        ),
    )(*ops1)

    ops2 = (x1, img, mask3, Wt, bt.reshape(1, P)) + _layer_ops(layers[1])
    in_specs2 = [_batched(x1), _batched(img), _batched(mask3)] + [
        _full(a) for a in ops2[3:]
    ]
    partials = pl.pallas_call(
        _tc_body2,
        grid=(B,),
        in_specs=in_specs2,
        out_specs=pl.BlockSpec((1, 1, 1), lambda b: (b, 0, 0)),
        out_shape=jax.ShapeDtypeStruct((B, 1, 1), jnp.float32),
        compiler_params=pltpu.CompilerParams(
            dimension_semantics=("arbitrary",),
        ),
    )(*ops2)

    return jnp.sum(partials) / (B * NUM_MASKED * P) / NUM_MASKED
